# baseline (speedup 1.0000x reference)
import functools

import jax
import jax.numpy as jnp
from jax import lax
from jax.experimental import pallas as pl
from jax.experimental.pallas import tpu as pltpu
from jax.experimental.pallas import tpu_sc as plsc

_NUM_EMB = 100000
_EMB_DIM = 50
_ROW_PAD = 56
_BATCH = 4096

_info = plsc.get_sparse_core_info()
_NC = _info.num_cores
_NS = _info.num_subcores
_NW = _NC * _NS
_BPW = _BATCH // _NW

_mesh = plsc.VectorSubcoreMesh(core_axis_name="c", subcore_axis_name="s")


@functools.partial(
    pl.kernel,
    mesh=_mesh,
    out_type=jax.ShapeDtypeStruct((_BATCH, _ROW_PAD), jnp.float32),
    scratch_types=[
        pltpu.VMEM((2, _BPW), jnp.int32),
        pltpu.VMEM((_BPW,), jnp.int32),
        pltpu.VMEM((_BPW, _EMB_DIM), jnp.float32),
        pltpu.VMEM((_BPW, _EMB_DIM), jnp.float32),
        pltpu.VMEM((_BPW, _ROW_PAD), jnp.float32),
        pltpu.SemaphoreType.DMA,
    ],
    compiler_params=pltpu.CompilerParams(
        use_tc_tiling_on_sc=False,
        disable_bounds_checks=True,
        needs_layout_passes=False,
    ),
)
def _gather_rows(pairs_hbm, off_hbm, table_hbm, out_hbm,
                 pairs_v, off_v, raw_a, raw_b, out_v, sem):
    wid = lax.axis_index("s") * _NC + lax.axis_index("c")
    base = wid * _BPW
    pbase = 2 * base
    pltpu.sync_copy(pairs_hbm.at[pl.ds(pbase, _BPW)], pairs_v.at[0])
    pltpu.sync_copy(pairs_hbm.at[pl.ds(pbase + _BPW, _BPW)], pairs_v.at[1])
    pltpu.sync_copy(off_hbm.at[pl.ds(base, _BPW)], off_v)
    c0 = pltpu.async_copy(table_hbm.at[pairs_v.at[0]], raw_a, sem)
    c1 = pltpu.async_copy(table_hbm.at[pairs_v.at[1]], raw_b, sem)
    c0.wait()
    c1.wait()
    for rb in range(_BPW // 16):
        r_vec = rb * 16 + lax.iota(jnp.int32, 16)
        half = rb // 4
        raw = raw_a if half == 0 else raw_b
        p0 = 100 * (r_vec - 64 * half) + off_v[pl.ds(rb * 16, 16)]
        for c in range(_EMB_DIM):
            p = p0 + c
            q = (p * 9363) >> 19
            rem = p - q * 56
            vals = plsc.load_gather(raw, [q, rem])
            plsc.store_scatter(
                out_v, [r_vec, jnp.full((16,), c, jnp.int32)], vals)
    pltpu.sync_copy(out_v, out_hbm.at[pl.ds(base, _BPW)])


def kernel(cats, conts, prefix_table):
    idx = cats[:, 0].astype(jnp.int32)
    start = idx * _ROW_PAD
    j0 = start // _EMB_DIM
    off = start - j0 * _EMB_DIM
    pairs = jnp.stack([j0, j0 + 1], axis=1).reshape(-1)
    padded = _gather_rows(pairs, off, prefix_table)
    return padded[:, :_EMB_DIM]


# speedup vs baseline: 1.0041x; 1.0041x over previous
import functools

import jax
import jax.numpy as jnp
from jax import lax
from jax.experimental import pallas as pl
from jax.experimental.pallas import tpu as pltpu
from jax.experimental.pallas import tpu_sc as plsc

_NUM_EMB = 100000
_EMB_DIM = 50
_ROW_PAD = 56
_BATCH = 4096

_info = plsc.get_sparse_core_info()
_NC = _info.num_cores
_NS = _info.num_subcores
_NW = _NC * _NS
_BPW = _BATCH // _NW

_mesh = plsc.VectorSubcoreMesh(core_axis_name="c", subcore_axis_name="s")


@functools.partial(
    pl.kernel,
    mesh=_mesh,
    out_type=jax.ShapeDtypeStruct((_BATCH, _ROW_PAD), jnp.float32),
    scratch_types=[
        pltpu.VMEM((2, _BPW), jnp.int32),
        pltpu.VMEM((_BPW,), jnp.int32),
        pltpu.VMEM((_BPW, _EMB_DIM), jnp.float32),
        pltpu.VMEM((_BPW, _EMB_DIM), jnp.float32),
        pltpu.VMEM((_BPW, _ROW_PAD), jnp.float32),
        pltpu.SemaphoreType.DMA,
    ],
    compiler_params=pltpu.CompilerParams(
        use_tc_tiling_on_sc=False,
        disable_bounds_checks=True,
        needs_layout_passes=False,
    ),
)
def _gather_rows(pairs_hbm, off_hbm, table_hbm, out_hbm,
                 pairs_v, off_v, raw_a, raw_b, out_v, sem):
    wid = lax.axis_index("s") * _NC + lax.axis_index("c")
    base = wid * _BPW
    pbase = 2 * base
    pltpu.sync_copy(pairs_hbm.at[pl.ds(pbase, _BPW)], pairs_v.at[0])
    pltpu.sync_copy(pairs_hbm.at[pl.ds(pbase + _BPW, _BPW)], pairs_v.at[1])
    pltpu.sync_copy(off_hbm.at[pl.ds(base, _BPW)], off_v)
    c0 = pltpu.async_copy(table_hbm.at[pairs_v.at[0]], raw_a, sem)
    c1 = pltpu.async_copy(table_hbm.at[pairs_v.at[1]], raw_b, sem)
    c0.wait()
    c1.wait()
    for rb in range(_BPW // 16):
        r_vec = rb * 16 + lax.iota(jnp.int32, 16)
        half = rb // 4
        raw = raw_a if half == 0 else raw_b
        p0 = 100 * (r_vec - 64 * half) + off_v[pl.ds(rb * 16, 16)]
        for c in range(_EMB_DIM):
            p = p0 + c
            q = (p * 9363) >> 19
            rem = p - q * 56
            vals = plsc.load_gather(raw, [q, rem])
            plsc.store_scatter(
                out_v, [r_vec, jnp.full((16,), c, jnp.int32)], vals)
    pltpu.sync_copy(out_v, out_hbm.at[pl.ds(base, _BPW)])


def kernel(cats, conts, prefix_table):
    idx = cats[:, 0].astype(jnp.int32)
    start = idx * _ROW_PAD
    j0 = start // _EMB_DIM
    off = start - j0 * _EMB_DIM
    pairs = jnp.stack([j0, j0 + 1], axis=1).reshape(-1)
    padded = _gather_rows(pairs, off, prefix_table)
    one = conts[0, 0] * 0.0 + 1.0
    return padded[:, :_EMB_DIM] * one


# speedup vs baseline: 2.3814x; 2.3716x over previous
import functools

import jax
import jax.numpy as jnp
from jax import lax
from jax.experimental import pallas as pl
from jax.experimental.pallas import tpu as pltpu
from jax.experimental.pallas import tpu_sc as plsc

_NUM_EMB = 100000
_EMB_DIM = 50
_ROW_PAD = 56
_BATCH = 4096

_info = plsc.get_sparse_core_info()
_NC = _info.num_cores
_NS = _info.num_subcores
_NW = _NC * _NS
_BPW = _BATCH // _NW

_mesh = plsc.VectorSubcoreMesh(core_axis_name="c", subcore_axis_name="s")


@functools.partial(
    pl.kernel,
    mesh=_mesh,
    out_type=jax.ShapeDtypeStruct((_BATCH, _ROW_PAD), jnp.float32),
    scratch_types=[
        pltpu.VMEM((_BPW,), jnp.int32),
        pltpu.VMEM((_EMB_DIM, _BPW), jnp.int32),
        pltpu.VMEM((_EMB_DIM, _BPW), jnp.float32),
        pltpu.VMEM((_BPW, _ROW_PAD), jnp.float32),
        pltpu.SemaphoreType.DMA,
    ],
    compiler_params=pltpu.CompilerParams(
        use_tc_tiling_on_sc=False,
        disable_bounds_checks=True,
        needs_layout_passes=False,
    ),
)
def _gather_rows(idx_hbm, tcol_hbm, out_hbm, idx_v, idxc_v, raw_v, out_v, sem):
    wid = lax.axis_index("s") * _NC + lax.axis_index("c")
    base = wid * _BPW
    pltpu.sync_copy(idx_hbm.at[pl.ds(base, _BPW)], idx_v)
    for g in range(_BPW // 16):
        v = idx_v[pl.ds(g * 16, 16)]
        for c in range(_EMB_DIM):
            idxc_v[c, pl.ds(g * 16, 16)] = v + c * _NUM_EMB
    copies = [
        pltpu.async_copy(tcol_hbm.at[idxc_v.at[c]], raw_v.at[c], sem)
        for c in range(_EMB_DIM)
    ]
    for cp in copies:
        cp.wait()
    for g in range(_BPW // 16):
        r_vec = g * 16 + lax.iota(jnp.int32, 16)
        for c in range(_EMB_DIM):
            vals = raw_v[c, pl.ds(g * 16, 16)]
            plsc.store_scatter(
                out_v, [r_vec, jnp.full((16,), c, jnp.int32)], vals)
    pltpu.sync_copy(out_v, out_hbm.at[pl.ds(base, _BPW)])


def kernel(cats, conts, prefix_table):
    idx = cats[:, 0].astype(jnp.int32)
    tcol = jnp.reshape(prefix_table.T, (-1,))
    padded = _gather_rows(idx, tcol)
    one = conts[0, 0] * 0.0 + 1.0
    return padded[:, :_EMB_DIM] * one
